# baseline (speedup 1.0000x reference)
import functools

import jax
import jax.numpy as jnp
from jax import lax
from jax.experimental import pallas as pl
from jax.experimental.pallas import tpu as pltpu
from jax.experimental.pallas import tpu_sc as plsc

_NC = 2
_NS = 16
_CHUNK_ROWS = 64


def kernel(x, pos_emb):
    T = x.shape[1]
    D = pos_emb.shape[1]
    nw = _NC * _NS
    rows_per_w = T // nw
    n_chunks = rows_per_w // _CHUNK_ROWS
    pos = jnp.arange(T, dtype=jnp.int32)
    mesh = plsc.VectorSubcoreMesh(core_axis_name="c", subcore_axis_name="s")

    @functools.partial(
        pl.kernel,
        mesh=mesh,
        out_type=jax.ShapeDtypeStruct((T, D), pos_emb.dtype),
        scratch_types=[
            pltpu.VMEM((_CHUNK_ROWS,), jnp.int32),
            pltpu.VMEM((_CHUNK_ROWS, D), pos_emb.dtype),
            pltpu.SemaphoreType.DMA,
        ],
    )
    def sc_gather(emb_hbm, idx_hbm, out_hbm, idx_v, rows_v, sem):
        wid = lax.axis_index("s") * _NC + lax.axis_index("c")
        base = wid * rows_per_w
        for j in range(n_chunks):
            r = base + j * _CHUNK_ROWS
            pltpu.sync_copy(idx_hbm.at[pl.ds(r, _CHUNK_ROWS)], idx_v)
            pltpu.async_copy(emb_hbm.at[idx_v], rows_v, sem).wait()
            pltpu.sync_copy(rows_v, out_hbm.at[pl.ds(r, _CHUNK_ROWS), :])

    return sc_gather(pos_emb, pos).reshape(1, T, D)


# speedup vs baseline: 1.0597x; 1.0597x over previous
import functools

import jax
import jax.numpy as jnp
from jax import lax
from jax.experimental import pallas as pl
from jax.experimental.pallas import tpu as pltpu
from jax.experimental.pallas import tpu_sc as plsc

_NC = 2
_NS = 16
_CHUNK_ROWS = 16
_NBUF = 7


def kernel(x, pos_emb):
    T = x.shape[1]
    D = pos_emb.shape[1]
    nw = _NC * _NS
    rows_per_w = T // nw
    n_chunks = rows_per_w // _CHUNK_ROWS
    mesh = plsc.VectorSubcoreMesh(core_axis_name="c", subcore_axis_name="s")

    @functools.partial(
        pl.kernel,
        mesh=mesh,
        out_type=jax.ShapeDtypeStruct((T, D), pos_emb.dtype),
        scratch_types=[
            pltpu.VMEM((_NBUF, _CHUNK_ROWS, D), pos_emb.dtype),
            pltpu.SemaphoreType.DMA((_NBUF,)),
            pltpu.SemaphoreType.DMA((_NBUF,)),
        ],
    )
    def sc_copy(emb_hbm, out_hbm, bufs, sem_in, sem_out):
        wid = lax.axis_index("s") * _NC + lax.axis_index("c")
        base = wid * rows_per_w

        def start_in(j):
            r = base + j * _CHUNK_ROWS
            return pltpu.async_copy(
                emb_hbm.at[pl.ds(r, _CHUNK_ROWS), :],
                bufs.at[j % _NBUF],
                sem_in.at[j % _NBUF],
            )

        def start_out(j):
            r = base + j * _CHUNK_ROWS
            return pltpu.async_copy(
                bufs.at[j % _NBUF],
                out_hbm.at[pl.ds(r, _CHUNK_ROWS), :],
                sem_out.at[j % _NBUF],
            )

        ins = [None] * n_chunks
        outs = [None] * n_chunks
        for j in range(min(_NBUF, n_chunks)):
            ins[j] = start_in(j)
        for j in range(n_chunks):
            ins[j].wait()
            outs[j] = start_out(j)
            nxt = j + _NBUF
            if nxt < n_chunks:
                outs[j].wait()
                ins[nxt] = start_in(nxt)
        for j in range(max(0, n_chunks - _NBUF), n_chunks):
            outs[j].wait()

    return sc_copy(pos_emb).reshape(1, T, D)
